# baseline (speedup 1.0000x reference)
import functools

import jax
import jax.numpy as jnp
from jax import lax
from jax.experimental import pallas as pl
from jax.experimental.pallas import tpu as pltpu
from jax.experimental.pallas import tpu_sc as plsc

B, P, K = 2, 10000, 16
C = 128
COUT = 128
H = W = 100
BP = B * P
NC, NS, L = 2, 16, 16
NW = NC * NS
CP = 8
CPK = CP * K
NCHUNK = BP // CP
TPWP = 80
NBUF = 5
EIROWS = NW * TPWP


def _tc_matmul(x_flat, W_conv, b_row):
    BLK = 2000

    def body(x_ref, w_ref, b_ref, a_ref, n_ref):
        w = w_ref[...]
        w1 = w[:, :C]
        w2 = w[:, C:]
        xb = x_ref[...]
        dn = (((1,), (1,)), ((), ()))
        a_ref[...] = lax.dot_general(
            xb, w1 - w2, dn, preferred_element_type=jnp.float32) + b_ref[...]
        n_ref[...] = lax.dot_general(
            xb, w2, dn, preferred_element_type=jnp.float32)

    return pl.pallas_call(
        body,
        grid=(BP // BLK,),
        in_specs=[
            pl.BlockSpec((BLK, C), lambda i: (i, 0)),
            pl.BlockSpec((COUT, 2 * C), lambda i: (0, 0)),
            pl.BlockSpec((1, COUT), lambda i: (0, 0)),
        ],
        out_specs=[
            pl.BlockSpec((BLK, COUT), lambda i: (i, 0)),
            pl.BlockSpec((BLK, COUT), lambda i: (i, 0)),
        ],
        out_shape=[
            jax.ShapeDtypeStruct((BP, COUT), jnp.float32),
            jax.ShapeDtypeStruct((BP, COUT), jnp.float32),
        ],
    )(x_flat, W_conv, b_row)


def _sc_gather_max(nv, ei2d):
    mesh = plsc.VectorSubcoreMesh(core_axis_name="c", subcore_axis_name="s")
    scratch = (
        [pltpu.VMEM((TPWP, CPK), jnp.int32)]
        + [pltpu.VMEM((CPK, COUT), jnp.float32) for _ in range(NBUF)]
        + [pltpu.VMEM((CP, COUT), jnp.float32) for _ in range(NBUF)]
        + [pltpu.SemaphoreType.DMA for _ in range(2 * NBUF)]
    )

    @functools.partial(
        pl.kernel,
        out_type=jax.ShapeDtypeStruct((BP, COUT), jnp.float32),
        mesh=mesh,
        scratch_types=scratch,
    )
    def k(nv_hbm, ei_hbm, out_hbm, idxs, *bufs):
        g = bufs[0:NBUF]
        o = bufs[NBUF:2 * NBUF]
        sg = bufs[2 * NBUF:3 * NBUF]
        so = bufs[3 * NBUF:4 * NBUF]
        wid = lax.axis_index("s") * NC + lax.axis_index("c")
        lo = wid * TPWP
        nw = jnp.clip(NCHUNK - lo, 0, TPWP)

        pltpu.sync_copy(ei_hbm.at[pl.ds(lo, TPWP)], idxs)

        def fire(b, c):
            pltpu.async_copy(nv_hbm.at[idxs.at[c]], g[b], sg[b])

        for b in range(NBUF):
            @pl.when(b < nw)
            def _(b=b):
                fire(b, b)

        def outer(t, carry):
            base = t * NBUF
            for b in range(NBUF):
                c = base + b

                @pl.when(c < nw)
                def _(b=b, c=c):
                    pltpu.make_async_copy(nv_hbm.at[idxs.at[c]], g[b], sg[b]).wait()

                    @pl.when(c >= NBUF)
                    def _():
                        pltpu.make_async_copy(
                            o[b], out_hbm.at[pl.ds(0, CP)], so[b]).wait()

                    def point_body(p, c2):
                        rbase = p * K
                        for cb in range(COUT // L):
                            sl = pl.ds(cb * L, L)
                            m = g[b][rbase, sl]
                            for j in range(1, K):
                                m = jnp.maximum(m, g[b][rbase + j, sl])
                            o[b][p, sl] = m
                        return c2

                    lax.fori_loop(0, CP, point_body, 0)
                    pltpu.async_copy(
                        o[b], out_hbm.at[pl.ds((lo + c) * CP, CP)], so[b])

                    @pl.when(c + NBUF < nw)
                    def _(b=b, c=c):
                        fire(b, c + NBUF)

            return carry

        lax.fori_loop(0, TPWP // NBUF, outer, 0)

        for b in range(NBUF):
            pltpu.make_async_copy(o[b], out_hbm.at[pl.ds(0, CP)], so[b]).wait()

    return k(nv, ei2d)


def _tc_finish(m, a):
    BLK = 2000

    def body(m_ref, a_ref, o_ref):
        o_ref[...] = jnp.maximum(m_ref[...] + a_ref[...], 0.0)

    return pl.pallas_call(
        body,
        grid=(BP // BLK,),
        in_specs=[
            pl.BlockSpec((BLK, COUT), lambda i: (i, 0)),
            pl.BlockSpec((BLK, COUT), lambda i: (i, 0)),
        ],
        out_specs=pl.BlockSpec((BLK, COUT), lambda i: (i, 0)),
        out_shape=jax.ShapeDtypeStruct((BP, COUT), jnp.float32),
    )(m, a)


def kernel(x, edge_index, size, W_conv, b_conv):
    del size
    x_flat = x.reshape(BP, C)
    offs = (jnp.arange(B, dtype=edge_index.dtype) * P).reshape(B, 1, 1)
    ei = (edge_index + offs).reshape(BP * K)
    ei = jnp.concatenate(
        [ei, jnp.zeros((EIROWS * CPK - BP * K,), dtype=ei.dtype)])
    ei2d = ei.reshape(EIROWS, CPK)
    a_mat, nv = _tc_matmul(x_flat, W_conv, b_conv.reshape(1, COUT))
    m = _sc_gather_max(nv, ei2d)
    y3 = _tc_finish(m, a_mat).reshape(B, P, COUT)
    return jnp.transpose(y3, (0, 2, 1)).reshape(B, COUT, H, W)


# speedup vs baseline: 1.1048x; 1.1048x over previous
import functools

import jax
import jax.numpy as jnp
from jax import lax
from jax.experimental import pallas as pl
from jax.experimental.pallas import tpu as pltpu
from jax.experimental.pallas import tpu_sc as plsc

B, P, K = 2, 10000, 16
C = 128
COUT = 128
H = W = 100
BP = B * P
NC, NS, L = 2, 16, 16
NW = NC * NS
CP = 8
CPK = CP * K
NCHUNK = BP // CP
TPWP = 80
NBUF = 4
EIROWS = NW * TPWP


def _tc_matmul(x_flat, W_conv):
    BLK = 2000

    def body(x_ref, w_ref, n_ref):
        w2 = w_ref[...][:, C:]
        dn = (((1,), (1,)), ((), ()))
        n_ref[...] = lax.dot_general(
            x_ref[...], w2, dn, preferred_element_type=jnp.float32)

    return pl.pallas_call(
        body,
        grid=(BP // BLK,),
        in_specs=[
            pl.BlockSpec((BLK, C), lambda i: (i, 0)),
            pl.BlockSpec((COUT, 2 * C), lambda i: (0, 0)),
        ],
        out_specs=pl.BlockSpec((BLK, COUT), lambda i: (i, 0)),
        out_shape=jax.ShapeDtypeStruct((BP, COUT), jnp.float32),
    )(x_flat, W_conv)


def _sc_gather_max(nv, ei2d):
    mesh = plsc.VectorSubcoreMesh(core_axis_name="c", subcore_axis_name="s")
    scratch = (
        [pltpu.VMEM((TPWP, CPK), jnp.int32)]
        + [pltpu.VMEM((CPK, COUT), jnp.float32) for _ in range(NBUF)]
        + [pltpu.VMEM((CP, COUT), jnp.float32) for _ in range(NBUF)]
        + [pltpu.SemaphoreType.DMA for _ in range(2 * NBUF)]
    )

    @functools.partial(
        pl.kernel,
        out_type=jax.ShapeDtypeStruct((BP, COUT), jnp.float32),
        mesh=mesh,
        scratch_types=scratch,
    )
    def k(nv_hbm, ei_hbm, out_hbm, idxs, *bufs):
        g = bufs[0:NBUF]
        o = bufs[NBUF:2 * NBUF]
        sg = bufs[2 * NBUF:3 * NBUF]
        so = bufs[3 * NBUF:4 * NBUF]
        wid = lax.axis_index("s") * NC + lax.axis_index("c")
        lo = wid * TPWP
        nw = jnp.clip(NCHUNK - lo, 0, TPWP)

        pltpu.sync_copy(ei_hbm.at[pl.ds(lo, TPWP)], idxs)

        def fire(b, c):
            pltpu.async_copy(nv_hbm.at[idxs.at[c]], g[b], sg[b])

        for b in range(NBUF):
            @pl.when(b < nw)
            def _(b=b):
                fire(b, b)

        def outer(t, carry):
            base = t * NBUF
            for b in range(NBUF):
                c = base + b

                @pl.when(c < nw)
                def _(b=b, c=c):
                    pltpu.make_async_copy(nv_hbm.at[idxs.at[c]], g[b], sg[b]).wait()

                    @pl.when(c >= NBUF)
                    def _():
                        pltpu.make_async_copy(
                            o[b], out_hbm.at[pl.ds(0, CP)], so[b]).wait()

                    def point_body(p, c2):
                        rbase = p * K
                        for cb in range(COUT // L):
                            sl = pl.ds(cb * L, L)
                            m = g[b][rbase, sl]
                            for j in range(1, K):
                                m = jnp.maximum(m, g[b][rbase + j, sl])
                            o[b][p, sl] = m
                        return c2

                    lax.fori_loop(0, CP, point_body, 0)
                    pltpu.async_copy(
                        o[b], out_hbm.at[pl.ds((lo + c) * CP, CP)], so[b])

                    @pl.when(c + NBUF < nw)
                    def _(b=b, c=c):
                        fire(b, c + NBUF)

            return carry

        lax.fori_loop(0, TPWP // NBUF, outer, 0)

        for b in range(NBUF):
            pltpu.make_async_copy(o[b], out_hbm.at[pl.ds(0, CP)], so[b]).wait()

    return k(nv, ei2d)


def _tc_finish(m, x_flat, W_conv, b_row):
    BLK = 2000

    def body(m_ref, x_ref, w_ref, b_ref, o_ref):
        w = w_ref[...]
        dw = w[:, :C] - w[:, C:]
        dn = (((1,), (1,)), ((), ()))
        a = lax.dot_general(
            x_ref[...], dw, dn, preferred_element_type=jnp.float32)
        o_ref[...] = jnp.maximum(a + b_ref[...] + m_ref[...], 0.0)

    return pl.pallas_call(
        body,
        grid=(BP // BLK,),
        in_specs=[
            pl.BlockSpec((BLK, COUT), lambda i: (i, 0)),
            pl.BlockSpec((BLK, C), lambda i: (i, 0)),
            pl.BlockSpec((COUT, 2 * C), lambda i: (0, 0)),
            pl.BlockSpec((1, COUT), lambda i: (0, 0)),
        ],
        out_specs=pl.BlockSpec((BLK, COUT), lambda i: (i, 0)),
        out_shape=jax.ShapeDtypeStruct((BP, COUT), jnp.float32),
    )(m, x_flat, W_conv, b_row)


def kernel(x, edge_index, size, W_conv, b_conv):
    del size
    x_flat = x.reshape(BP, C)
    offs = (jnp.arange(B, dtype=edge_index.dtype) * P).reshape(B, 1, 1)
    ei = (edge_index + offs).reshape(BP * K)
    ei = jnp.concatenate(
        [ei, jnp.zeros((EIROWS * CPK - BP * K,), dtype=ei.dtype)])
    ei2d = ei.reshape(EIROWS, CPK)
    nv = _tc_matmul(x_flat, W_conv)
    m = _sc_gather_max(nv, ei2d)
    y3 = _tc_finish(m, x_flat, W_conv, b_conv.reshape(1, COUT))
    y3 = y3.reshape(B, P, COUT)
    return jnp.transpose(y3, (0, 2, 1)).reshape(B, COUT, H, W)


# speedup vs baseline: 1.1148x; 1.0090x over previous
import functools

import jax
import jax.numpy as jnp
from jax import lax
from jax.experimental import pallas as pl
from jax.experimental.pallas import tpu as pltpu
from jax.experimental.pallas import tpu_sc as plsc

B, P, K = 2, 10000, 16
C = 128
COUT = 128
H = W = 100
BP = B * P
NC, NS, L = 2, 16, 16
NW = NC * NS
CP = 8
CPK = CP * K
NCHUNK = BP // CP
TPWP = 80
NBUF = 4


def _tc_matmul(x_flat, W_conv):
    BLK = 2000

    def body(x_ref, w_ref, n_ref):
        w2 = w_ref[...][:, C:]
        dn = (((1,), (1,)), ((), ()))
        n_ref[...] = lax.dot_general(
            x_ref[...], w2, dn, preferred_element_type=jnp.float32)

    return pl.pallas_call(
        body,
        grid=(BP // BLK,),
        in_specs=[
            pl.BlockSpec((BLK, C), lambda i: (i, 0)),
            pl.BlockSpec((COUT, 2 * C), lambda i: (0, 0)),
        ],
        out_specs=pl.BlockSpec((BLK, COUT), lambda i: (i, 0)),
        out_shape=jax.ShapeDtypeStruct((BP, COUT), jnp.float32),
    )(x_flat, W_conv)


def _sc_gather_max(nv, ei2d):
    mesh = plsc.VectorSubcoreMesh(core_axis_name="c", subcore_axis_name="s")
    scratch = (
        [pltpu.VMEM((TPWP * CPK,), jnp.int32)]
        + [pltpu.VMEM((CPK, COUT), jnp.float32) for _ in range(NBUF)]
        + [pltpu.VMEM((CP, COUT), jnp.float32) for _ in range(NBUF)]
        + [pltpu.SemaphoreType.DMA for _ in range(2 * NBUF)]
    )

    @functools.partial(
        pl.kernel,
        out_type=jax.ShapeDtypeStruct((BP, COUT), jnp.float32),
        mesh=mesh,
        scratch_types=scratch,
    )
    def k(nv_hbm, ei_hbm, out_hbm, idxs, *bufs):
        g = bufs[0:NBUF]
        o = bufs[NBUF:2 * NBUF]
        sg = bufs[2 * NBUF:3 * NBUF]
        so = bufs[3 * NBUF:4 * NBUF]
        wid = lax.axis_index("s") * NC + lax.axis_index("c")
        lo = (NCHUNK * wid) // NW
        nw = (NCHUNK * (wid + 1)) // NW - lo
        sbase = jnp.minimum(lo, NCHUNK - TPWP)
        shift = lo - sbase

        pltpu.sync_copy(ei_hbm.at[pl.ds(sbase * CPK, TPWP * CPK)], idxs)

        def fire(b, c):
            pltpu.async_copy(
                nv_hbm.at[idxs.at[pl.ds((c + shift) * CPK, CPK)]], g[b], sg[b])

        for b in range(NBUF):
            @pl.when(b < nw)
            def _(b=b):
                fire(b, b)

        def outer(t, carry):
            base = t * NBUF
            for b in range(NBUF):
                c = base + b

                @pl.when(c < nw)
                def _(b=b, c=c):
                    pltpu.make_async_copy(
                        nv_hbm.at[idxs.at[pl.ds(0, CPK)]], g[b], sg[b]).wait()

                    @pl.when(c >= NBUF)
                    def _():
                        pltpu.make_async_copy(
                            o[b], out_hbm.at[pl.ds(0, CP)], so[b]).wait()

                    def point_body(p, c2):
                        rbase = p * K
                        for cb in range(COUT // L):
                            sl = pl.ds(cb * L, L)
                            m = g[b][rbase, sl]
                            for j in range(1, K):
                                m = jnp.maximum(m, g[b][rbase + j, sl])
                            o[b][p, sl] = m
                        return c2

                    lax.fori_loop(0, CP, point_body, 0)
                    pltpu.async_copy(
                        o[b], out_hbm.at[pl.ds((lo + c) * CP, CP)], so[b])

                    @pl.when(c + NBUF < nw)
                    def _(b=b, c=c):
                        fire(b, c + NBUF)

            return carry

        lax.fori_loop(0, TPWP // NBUF, outer, 0)

        for b in range(NBUF):
            pltpu.make_async_copy(o[b], out_hbm.at[pl.ds(0, CP)], so[b]).wait()

    return k(nv, ei2d)


def _tc_finish(m, x_flat, W_conv, b_row):
    BLK = 2000

    def body(m_ref, x_ref, w_ref, b_ref, o_ref):
        w = w_ref[...]
        dw = w[:, :C] - w[:, C:]
        dn = (((1,), (1,)), ((), ()))
        a = lax.dot_general(
            x_ref[...], dw, dn, preferred_element_type=jnp.float32)
        o_ref[...] = jnp.maximum(a + b_ref[...] + m_ref[...], 0.0)

    return pl.pallas_call(
        body,
        grid=(BP // BLK,),
        in_specs=[
            pl.BlockSpec((BLK, COUT), lambda i: (i, 0)),
            pl.BlockSpec((BLK, C), lambda i: (i, 0)),
            pl.BlockSpec((COUT, 2 * C), lambda i: (0, 0)),
            pl.BlockSpec((1, COUT), lambda i: (0, 0)),
        ],
        out_specs=pl.BlockSpec((BLK, COUT), lambda i: (i, 0)),
        out_shape=jax.ShapeDtypeStruct((BP, COUT), jnp.float32),
    )(m, x_flat, W_conv, b_row)


def kernel(x, edge_index, size, W_conv, b_conv):
    del size
    x_flat = x.reshape(BP, C)
    offs = (jnp.arange(B, dtype=edge_index.dtype) * P).reshape(B, 1, 1)
    ei = (edge_index + offs).reshape(BP * K)
    nv = _tc_matmul(x_flat, W_conv)
    m = _sc_gather_max(nv, ei)
    y3 = _tc_finish(m, x_flat, W_conv, b_conv.reshape(1, COUT))
    y3 = y3.reshape(B, P, COUT)
    return jnp.transpose(y3, (0, 2, 1)).reshape(B, COUT, H, W)
